# baseline (speedup 1.0000x reference)
import functools

import jax
import jax.numpy as jnp
from jax import lax
from jax.experimental import pallas as pl
from jax.experimental.pallas import tpu as pltpu
from jax.experimental.pallas import tpu_sc as plsc

B, S, D, F, E = 2, 8192, 768, 2048, 64
N = B * S
TM = 128
FBLK = 512
NF = F // FBLK
T = N // TM
P = T + E
NP = P * TM
RB = 512
NB = N // RB
SQRT1_2 = 0.7071067811865476


def _router_body(x_ref, sw_ref, sb_ref, routes_ref, pmax_ref):
    xb = x_ref[...]
    logits = lax.dot_general(xb, sw_ref[...], (((1,), (1,)), ((), ())),
                             preferred_element_type=jnp.float32)
    logits = logits + sb_ref[0][None, :]
    m = jnp.max(logits, axis=-1, keepdims=True)
    denom = jnp.sum(jnp.exp(logits - m), axis=-1)
    pmax_ref[0, 0, :] = 1.0 / denom
    ids = lax.broadcasted_iota(jnp.int32, logits.shape, 1)
    routes_ref[0, 0, :] = jnp.min(jnp.where(logits == m, ids, E), axis=-1)


def _ffn_body(eb_ref, x_ref, w1_ref, b1_ref, w2_ref, b2_ref, pr_ref, o_ref):
    f = pl.program_id(1)
    xb = x_ref[...]
    h = lax.dot_general(xb, w1_ref[0], (((1,), (1,)), ((), ())),
                        preferred_element_type=jnp.float32)
    h = h + b1_ref[0, 0][None, :]
    g = 0.5 * h * (1.0 + lax.erf(h * SQRT1_2))
    contrib = lax.dot_general(g, w2_ref[0], (((1,), (1,)), ((), ())),
                              preferred_element_type=jnp.float32)

    @pl.when(f == 0)
    def _():
        o_ref[...] = contrib + b2_ref[0, 0][None, :]

    @pl.when(f > 0)
    def _():
        o_ref[...] += contrib

    @pl.when(f == NF - 1)
    def _():
        o_ref[...] *= pr_ref[0, 0][:, None]


def _route_tokens(flat, switch_w, switch_b):
    routes2, pmax2 = pl.pallas_call(
        _router_body,
        grid=(NB,),
        in_specs=[
            pl.BlockSpec((RB, D), lambda i: (i, 0)),
            pl.BlockSpec((E, D), lambda i: (0, 0)),
            pl.BlockSpec((1, E), lambda i: (0, 0)),
        ],
        out_specs=[
            pl.BlockSpec((1, 1, RB), lambda i: (i, 0, 0)),
            pl.BlockSpec((1, 1, RB), lambda i: (i, 0, 0)),
        ],
        out_shape=[
            jax.ShapeDtypeStruct((NB, 1, RB), jnp.int32),
            jax.ShapeDtypeStruct((NB, 1, RB), jnp.float32),
        ],
    )(flat, switch_w, switch_b.reshape(1, E))
    return routes2.reshape(N), pmax2.reshape(N)


def _grouped_ffn(eblk, xg, w1, b1, w2, b2, pg2):
    grid_spec = pltpu.PrefetchScalarGridSpec(
        num_scalar_prefetch=1,
        grid=(P, NF),
        in_specs=[
            pl.BlockSpec((TM, D), lambda p, f, eb: (p, 0)),
            pl.BlockSpec((1, FBLK, D), lambda p, f, eb: (eb[p], f, 0)),
            pl.BlockSpec((1, 1, FBLK), lambda p, f, eb: (eb[p] * NF + f, 0, 0)),
            pl.BlockSpec((1, D, FBLK), lambda p, f, eb: (eb[p], 0, f)),
            pl.BlockSpec((1, 1, D), lambda p, f, eb: (eb[p], 0, 0)),
            pl.BlockSpec((1, 1, TM), lambda p, f, eb: (p, 0, 0)),
        ],
        out_specs=pl.BlockSpec((TM, D), lambda p, f, eb: (p, 0)),
    )
    return pl.pallas_call(
        _ffn_body,
        grid_spec=grid_spec,
        out_shape=jax.ShapeDtypeStruct((NP, D), jnp.float32),
        compiler_params=pltpu.CompilerParams(
            dimension_semantics=("arbitrary", "arbitrary")),
    )(eblk, xg, w1, b1.reshape(E * NF, 1, FBLK), w2, b2.reshape(E, 1, D), pg2)


def _sc_gather_rows(src, idx, n_out):
    info = plsc.get_sparse_core_info()
    nc, ns = info.num_cores, info.num_subcores
    nw = nc * ns
    per_w = n_out // nw
    ch = 128
    mesh = plsc.VectorSubcoreMesh(core_axis_name="c", subcore_axis_name="s")

    @functools.partial(
        pl.kernel, mesh=mesh,
        out_type=jax.ShapeDtypeStruct((n_out, D), jnp.float32),
        scratch_types=[
            pltpu.VMEM((ch,), jnp.int32),
            pltpu.VMEM((ch, D), jnp.float32),
            pltpu.SemaphoreType.DMA,
        ],
    )
    def gk(src_hbm, idx_hbm, out_hbm, idx_v, rows_v, sem):
        wid = lax.axis_index("s") * nc + lax.axis_index("c")
        base0 = wid * per_w
        for c in range(per_w // ch):
            base = base0 + c * ch
            pltpu.sync_copy(idx_hbm.at[pl.ds(base, ch)], idx_v)
            pltpu.async_copy(src_hbm.at[idx_v], rows_v, sem).wait()
            pltpu.sync_copy(rows_v, out_hbm.at[pl.ds(base, ch)])

    return gk(src, idx)


def kernel(x, switch_w, switch_b, w1, b1, w2, b2):
    flat = x.reshape(N, D)

    routes, pmax = _route_tokens(flat, switch_w, switch_b)

    perm = jnp.argsort(routes)
    counts = jnp.bincount(routes, length=E)
    offs = jnp.concatenate([jnp.zeros(1, jnp.int32),
                            jnp.cumsum(counts).astype(jnp.int32)])
    ptiles = (counts + TM - 1) // TM
    bcum = jnp.cumsum(ptiles)
    bstart = (bcum - ptiles).astype(jnp.int32)
    eblk = jnp.minimum(
        jnp.searchsorted(bcum, jnp.arange(P), side="right"), E - 1
    ).astype(jnp.int32)
    slots = jnp.arange(NP, dtype=jnp.int32)
    e_i = eblk[slots // TM]
    r = slots - TM * bstart[e_i]
    tok_q = offs[e_i] + r
    valid = r < counts[e_i]
    sidx = jnp.where(valid, perm[jnp.minimum(tok_q, N - 1)], 0).astype(jnp.int32)
    q = jnp.arange(N, dtype=jnp.int32)
    e_q = jnp.searchsorted(offs[1:], q, side="right").astype(jnp.int32)
    slot_q = TM * bstart[e_q] + (q - offs[e_q])
    inv = jnp.zeros(N, jnp.int32).at[perm].set(slot_q)

    xg = _sc_gather_rows(flat, sidx, NP)
    pg = pmax[sidx]

    ys = _grouped_ffn(eblk, xg, w1, b1, w2, b2, pg.reshape(P, 1, TM))

    final = _sc_gather_rows(ys, inv, N)
    return final.reshape(B, S, D)


# speedup vs baseline: 1.4673x; 1.4673x over previous
import functools

import jax
import jax.numpy as jnp
from jax import lax
from jax.experimental import pallas as pl
from jax.experimental.pallas import tpu as pltpu
from jax.experimental.pallas import tpu_sc as plsc

B, S, D, F, E = 2, 8192, 768, 2048, 64
N = B * S
TM = 128
FBLK = 512
NF = F // FBLK
T = N // TM
P = T + E
NP = P * TM
RB = 512
NB = N // RB
SQRT1_2 = 0.7071067811865476


def _router_body(x_ref, sw_ref, sb_ref, routes_ref, pmax_ref, rank_ref,
                 cnt_ref, xc_ref):
    xb = x_ref[...]
    logits = lax.dot_general(xb, sw_ref[...], (((1,), (1,)), ((), ())),
                             preferred_element_type=jnp.float32)
    logits = logits + sb_ref[0][None, :]
    m = jnp.max(logits, axis=-1, keepdims=True)
    denom = jnp.sum(jnp.exp(logits - m), axis=-1)
    pmax_ref[0, 0, :] = 1.0 / denom
    ids = lax.broadcasted_iota(jnp.int32, logits.shape, 1)
    routes = jnp.min(jnp.where(logits == m, ids, E), axis=-1)
    routes_ref[0, 0, :] = routes
    oh = (routes[:, None] == lax.broadcasted_iota(jnp.int32, (RB, E), 1))
    oh = oh.astype(jnp.float32)
    ti = lax.broadcasted_iota(jnp.int32, (RB, RB), 0)
    tj = lax.broadcasted_iota(jnp.int32, (RB, RB), 1)
    lt = (tj < ti).astype(jnp.float32)
    pre = lax.dot_general(lt, oh, (((1,), (0,)), ((), ())),
                          preferred_element_type=jnp.float32)
    rank_ref[0, 0, :] = jnp.sum(pre * oh, axis=1)
    cnt_ref[0, 0, :] = jnp.sum(oh, axis=0)
    xc_ref[...] = xb


def _slot_body(routes_ref, rank_ref, base_ref, slot_ref):
    r = routes_ref[0, 0, :]
    oh = (r[:, None] == lax.broadcasted_iota(jnp.int32, (RB, E), 1))
    base = jnp.sum(oh.astype(jnp.float32) * base_ref[0, 0, :][None, :], axis=1)
    slot_ref[0, 0, :] = (rank_ref[0, 0, :] + base).astype(jnp.int32)


def _ffn_body(eb_ref, x_ref, w1_ref, b1_ref, w2_ref, b2_ref, pr_ref, o_ref):
    f = pl.program_id(1)
    xb = x_ref[...]
    h = lax.dot_general(xb, w1_ref[0], (((1,), (1,)), ((), ())),
                        preferred_element_type=jnp.float32)
    h = h + b1_ref[0, 0][None, :]
    g = 0.5 * h * (1.0 + lax.erf(h * SQRT1_2))
    contrib = lax.dot_general(g, w2_ref[0], (((1,), (1,)), ((), ())),
                              preferred_element_type=jnp.float32)

    @pl.when(f == 0)
    def _():
        o_ref[...] = contrib + b2_ref[0, 0][None, :]

    @pl.when(f > 0)
    def _():
        o_ref[...] += contrib

    @pl.when(f == NF - 1)
    def _():
        o_ref[...] *= pr_ref[0, 0][:, None]


def _route_tokens(flat, switch_w, switch_b):
    return pl.pallas_call(
        _router_body,
        grid=(NB,),
        in_specs=[
            pl.BlockSpec((RB, D), lambda i: (i, 0)),
            pl.BlockSpec((E, D), lambda i: (0, 0)),
            pl.BlockSpec((1, E), lambda i: (0, 0)),
        ],
        out_specs=[
            pl.BlockSpec((1, 1, RB), lambda i: (i, 0, 0)),
            pl.BlockSpec((1, 1, RB), lambda i: (i, 0, 0)),
            pl.BlockSpec((1, 1, RB), lambda i: (i, 0, 0)),
            pl.BlockSpec((1, 1, E), lambda i: (i, 0, 0)),
            pl.BlockSpec((RB, D), lambda i: (i, 0)),
        ],
        out_shape=[
            jax.ShapeDtypeStruct((NB, 1, RB), jnp.int32),
            jax.ShapeDtypeStruct((NB, 1, RB), jnp.float32),
            jax.ShapeDtypeStruct((NB, 1, RB), jnp.float32),
            jax.ShapeDtypeStruct((NB, 1, E), jnp.float32),
            jax.ShapeDtypeStruct((N, D), jnp.float32),
        ],
    )(flat, switch_w, switch_b.reshape(1, E))


def _token_slots(routes2, rank2, base):
    slot2 = pl.pallas_call(
        _slot_body,
        grid=(NB,),
        in_specs=[
            pl.BlockSpec((1, 1, RB), lambda i: (i, 0, 0)),
            pl.BlockSpec((1, 1, RB), lambda i: (i, 0, 0)),
            pl.BlockSpec((1, 1, E), lambda i: (i, 0, 0)),
        ],
        out_specs=pl.BlockSpec((1, 1, RB), lambda i: (i, 0, 0)),
        out_shape=jax.ShapeDtypeStruct((NB, 1, RB), jnp.int32),
    )(routes2, rank2, base)
    return slot2.reshape(N)


def _grouped_ffn(eblk, xg, w1, b1, w2, b2, pg2):
    grid_spec = pltpu.PrefetchScalarGridSpec(
        num_scalar_prefetch=1,
        grid=(P, NF),
        in_specs=[
            pl.BlockSpec((TM, D), lambda p, f, eb: (p, 0)),
            pl.BlockSpec((1, FBLK, D), lambda p, f, eb: (eb[p], f, 0)),
            pl.BlockSpec((1, 1, FBLK), lambda p, f, eb: (eb[p] * NF + f, 0, 0)),
            pl.BlockSpec((1, D, FBLK), lambda p, f, eb: (eb[p], 0, f)),
            pl.BlockSpec((1, 1, D), lambda p, f, eb: (eb[p], 0, 0)),
            pl.BlockSpec((1, 1, TM), lambda p, f, eb: (p, 0, 0)),
        ],
        out_specs=pl.BlockSpec((TM, D), lambda p, f, eb: (p, 0)),
    )
    return pl.pallas_call(
        _ffn_body,
        grid_spec=grid_spec,
        out_shape=jax.ShapeDtypeStruct((NP, D), jnp.float32),
        compiler_params=pltpu.CompilerParams(
            dimension_semantics=("arbitrary", "arbitrary")),
    )(eblk, xg, w1, b1.reshape(E * NF, 1, FBLK), w2, b2.reshape(E, 1, D), pg2)


def _sc_gather_rows(src, idx, n_out):
    info = plsc.get_sparse_core_info()
    nc, ns = info.num_cores, info.num_subcores
    nw = nc * ns
    per_w = n_out // nw
    ch = 128
    mesh = plsc.VectorSubcoreMesh(core_axis_name="c", subcore_axis_name="s")

    @functools.partial(
        pl.kernel, mesh=mesh,
        out_type=jax.ShapeDtypeStruct((n_out, D), jnp.float32),
        scratch_types=[
            pltpu.VMEM((ch,), jnp.int32),
            pltpu.VMEM((ch, D), jnp.float32),
            pltpu.SemaphoreType.DMA,
        ],
    )
    def gk(src_hbm, idx_hbm, out_hbm, idx_v, rows_v, sem):
        wid = lax.axis_index("s") * nc + lax.axis_index("c")
        base0 = wid * per_w
        for c in range(per_w // ch):
            base = base0 + c * ch
            pltpu.sync_copy(idx_hbm.at[pl.ds(base, ch)], idx_v)
            pltpu.async_copy(src_hbm.at[idx_v], rows_v, sem).wait()
            pltpu.sync_copy(rows_v, out_hbm.at[pl.ds(base, ch)])

    return gk(src, idx)


def kernel(x, switch_w, switch_b, w1, b1, w2, b2):
    flat = x.reshape(N, D)

    routes2, pmax2, rank2, cnt2, xcopy = _route_tokens(flat, switch_w, switch_b)
    pmax = pmax2.reshape(N)

    cnt = cnt2.reshape(NB, E)
    cumpb = jnp.cumsum(cnt, axis=0) - cnt
    counts = cnt.sum(axis=0).astype(jnp.int32)
    ptiles = (counts + TM - 1) // TM
    bcum = jnp.cumsum(ptiles)
    pstart = (TM * (bcum - ptiles)).astype(jnp.float32)
    eblk = jnp.minimum(
        jnp.searchsorted(bcum, jnp.arange(P), side="right"), E - 1
    ).astype(jnp.int32)
    base = (pstart[None, :] + cumpb).reshape(NB, 1, E)

    slot = _token_slots(routes2, rank2, base)
    sidx = jnp.zeros(NP, jnp.int32).at[slot].set(jnp.arange(N, dtype=jnp.int32))
    pgp = jnp.zeros(NP, jnp.float32).at[slot].set(pmax)

    xg = _sc_gather_rows(xcopy, sidx, NP)

    ys = _grouped_ffn(eblk, xg, w1, b1, w2, b2, pgp.reshape(P, 1, TM))

    final = _sc_gather_rows(ys, slot, N)
    return final.reshape(B, S, D)


# speedup vs baseline: 1.8345x; 1.2502x over previous
import functools

import jax
import jax.numpy as jnp
from jax import lax
from jax.experimental import pallas as pl
from jax.experimental.pallas import tpu as pltpu
from jax.experimental.pallas import tpu_sc as plsc

B, S, D, F, E = 2, 8192, 768, 2048, 64
N = B * S
TM = 128
FBLK = 512
NF = F // FBLK
T = N // TM
P = T + E
NP = P * TM
RB = 512
NB = N // RB
SQRT1_2 = 0.7071067811865476


def _router_body(x_ref, sw_ref, sb_ref, routes_ref, pmax_ref, rank_ref,
                 cnt_ref, xc_ref):
    xb = x_ref[...]
    logits = lax.dot_general(xb, sw_ref[...], (((1,), (1,)), ((), ())),
                             preferred_element_type=jnp.float32)
    logits = logits + sb_ref[0][None, :]
    m = jnp.max(logits, axis=-1, keepdims=True)
    denom = jnp.sum(jnp.exp(logits - m), axis=-1)
    pmax_ref[0, 0, :] = 1.0 / denom
    ids = lax.broadcasted_iota(jnp.int32, logits.shape, 1)
    routes = jnp.min(jnp.where(logits == m, ids, E), axis=-1)
    routes_ref[0, 0, :] = routes
    oh = (routes[:, None] == lax.broadcasted_iota(jnp.int32, (RB, E), 1))
    oh = oh.astype(jnp.float32)
    ti = lax.broadcasted_iota(jnp.int32, (RB, RB), 0)
    tj = lax.broadcasted_iota(jnp.int32, (RB, RB), 1)
    lt = (tj < ti).astype(jnp.float32)
    pre = lax.dot_general(lt, oh, (((1,), (0,)), ((), ())),
                          preferred_element_type=jnp.float32)
    rank_ref[0, 0, :] = jnp.sum(pre * oh, axis=1)
    cnt_ref[0, 0, :] = jnp.sum(oh, axis=0)
    xc_ref[...] = xb


def _slot_body(routes_ref, rank_ref, base_ref, slot_ref):
    r = routes_ref[0, 0, :]
    oh = (r[:, None] == lax.broadcasted_iota(jnp.int32, (RB, E), 1))
    base = jnp.sum(oh.astype(jnp.float32) * base_ref[0, 0, :][None, :], axis=1)
    slot_ref[0, 0, :] = (rank_ref[0, 0, :] + base).astype(jnp.int32)


def _ffn_body(eb_ref, x_ref, w1_ref, b1_ref, w2_ref, b2_ref, pr_ref, o_ref):
    f = pl.program_id(1)
    xb = x_ref[...].astype(jnp.bfloat16)
    h = lax.dot_general(xb, w1_ref[0].astype(jnp.bfloat16),
                        (((1,), (1,)), ((), ())),
                        preferred_element_type=jnp.float32)
    h = h + b1_ref[0, 0][None, :]
    g = 0.5 * h * (1.0 + lax.erf(h * SQRT1_2))
    contrib = lax.dot_general(g.astype(jnp.bfloat16),
                              w2_ref[0].astype(jnp.bfloat16),
                              (((1,), (1,)), ((), ())),
                              preferred_element_type=jnp.float32)

    @pl.when(f == 0)
    def _():
        o_ref[...] = contrib + b2_ref[0, 0][None, :]

    @pl.when(f > 0)
    def _():
        o_ref[...] += contrib

    @pl.when(f == NF - 1)
    def _():
        o_ref[...] *= pr_ref[0, 0][:, None]


def _route_tokens(flat, switch_w, switch_b):
    return pl.pallas_call(
        _router_body,
        grid=(NB,),
        in_specs=[
            pl.BlockSpec((RB, D), lambda i: (i, 0)),
            pl.BlockSpec((E, D), lambda i: (0, 0)),
            pl.BlockSpec((1, E), lambda i: (0, 0)),
        ],
        out_specs=[
            pl.BlockSpec((1, 1, RB), lambda i: (i, 0, 0)),
            pl.BlockSpec((1, 1, RB), lambda i: (i, 0, 0)),
            pl.BlockSpec((1, 1, RB), lambda i: (i, 0, 0)),
            pl.BlockSpec((1, 1, E), lambda i: (i, 0, 0)),
            pl.BlockSpec((RB, D), lambda i: (i, 0)),
        ],
        out_shape=[
            jax.ShapeDtypeStruct((NB, 1, RB), jnp.int32),
            jax.ShapeDtypeStruct((NB, 1, RB), jnp.float32),
            jax.ShapeDtypeStruct((NB, 1, RB), jnp.float32),
            jax.ShapeDtypeStruct((NB, 1, E), jnp.float32),
            jax.ShapeDtypeStruct((N, D), jnp.float32),
        ],
    )(flat, switch_w, switch_b.reshape(1, E))


def _token_slots(routes2, rank2, base):
    slot2 = pl.pallas_call(
        _slot_body,
        grid=(NB,),
        in_specs=[
            pl.BlockSpec((1, 1, RB), lambda i: (i, 0, 0)),
            pl.BlockSpec((1, 1, RB), lambda i: (i, 0, 0)),
            pl.BlockSpec((1, 1, E), lambda i: (i, 0, 0)),
        ],
        out_specs=pl.BlockSpec((1, 1, RB), lambda i: (i, 0, 0)),
        out_shape=jax.ShapeDtypeStruct((NB, 1, RB), jnp.int32),
    )(routes2, rank2, base)
    return slot2.reshape(N)


def _grouped_ffn(eblk, xg, w1, b1, w2, b2, pg2):
    grid_spec = pltpu.PrefetchScalarGridSpec(
        num_scalar_prefetch=1,
        grid=(P, NF),
        in_specs=[
            pl.BlockSpec((TM, D), lambda p, f, eb: (p, 0)),
            pl.BlockSpec((1, FBLK, D), lambda p, f, eb: (eb[p], f, 0)),
            pl.BlockSpec((1, 1, FBLK), lambda p, f, eb: (eb[p] * NF + f, 0, 0)),
            pl.BlockSpec((1, D, FBLK), lambda p, f, eb: (eb[p], 0, f)),
            pl.BlockSpec((1, 1, D), lambda p, f, eb: (eb[p], 0, 0)),
            pl.BlockSpec((1, 1, TM), lambda p, f, eb: (p, 0, 0)),
        ],
        out_specs=pl.BlockSpec((TM, D), lambda p, f, eb: (p, 0)),
    )
    return pl.pallas_call(
        _ffn_body,
        grid_spec=grid_spec,
        out_shape=jax.ShapeDtypeStruct((NP, D), jnp.float32),
        compiler_params=pltpu.CompilerParams(
            dimension_semantics=("arbitrary", "arbitrary")),
    )(eblk, xg, w1, b1.reshape(E * NF, 1, FBLK), w2, b2.reshape(E, 1, D), pg2)


def _sc_gather_rows(src, idx, n_out):
    info = plsc.get_sparse_core_info()
    nc, ns = info.num_cores, info.num_subcores
    nw = nc * ns
    per_w = n_out // nw
    ch = 128
    mesh = plsc.VectorSubcoreMesh(core_axis_name="c", subcore_axis_name="s")

    @functools.partial(
        pl.kernel, mesh=mesh,
        out_type=jax.ShapeDtypeStruct((n_out, D), jnp.float32),
        scratch_types=[
            pltpu.VMEM((ch,), jnp.int32),
            pltpu.VMEM((ch, D), jnp.float32),
            pltpu.SemaphoreType.DMA,
        ],
    )
    def gk(src_hbm, idx_hbm, out_hbm, idx_v, rows_v, sem):
        wid = lax.axis_index("s") * nc + lax.axis_index("c")
        base0 = wid * per_w
        for c in range(per_w // ch):
            base = base0 + c * ch
            pltpu.sync_copy(idx_hbm.at[pl.ds(base, ch)], idx_v)
            pltpu.async_copy(src_hbm.at[idx_v], rows_v, sem).wait()
            pltpu.sync_copy(rows_v, out_hbm.at[pl.ds(base, ch)])

    return gk(src, idx)


def kernel(x, switch_w, switch_b, w1, b1, w2, b2):
    flat = x.reshape(N, D)

    routes2, pmax2, rank2, cnt2, xcopy = _route_tokens(flat, switch_w, switch_b)
    pmax = pmax2.reshape(N)

    cnt = cnt2.reshape(NB, E)
    cumpb = jnp.cumsum(cnt, axis=0) - cnt
    counts = cnt.sum(axis=0).astype(jnp.int32)
    ptiles = (counts + TM - 1) // TM
    bcum = jnp.cumsum(ptiles)
    pstart = (TM * (bcum - ptiles)).astype(jnp.float32)
    eblk = jnp.minimum(
        jnp.searchsorted(bcum, jnp.arange(P), side="right"), E - 1
    ).astype(jnp.int32)
    base = (pstart[None, :] + cumpb).reshape(NB, 1, E)

    slot = _token_slots(routes2, rank2, base)
    sidx = (jnp.arange(NP, dtype=jnp.int32) % N).at[slot].set(
        jnp.arange(N, dtype=jnp.int32), unique_indices=True,
        mode="promise_in_bounds")
    pgp = jnp.zeros(NP, jnp.float32).at[slot].set(
        pmax, unique_indices=True, mode="promise_in_bounds")

    xg = _sc_gather_rows(xcopy, sidx, NP)

    ys = _grouped_ffn(eblk, xg, w1, b1, w2, b2, pgp.reshape(P, 1, TM))

    final = _sc_gather_rows(ys, slot, N)
    return final.reshape(B, S, D)


# speedup vs baseline: 2.7780x; 1.5143x over previous
import functools

import jax
import jax.numpy as jnp
from jax import lax
from jax.experimental import pallas as pl
from jax.experimental.pallas import tpu as pltpu
from jax.experimental.pallas import tpu_sc as plsc

B, S, D, F, E = 2, 8192, 768, 2048, 64
N = B * S
TM = 128
FBLK = 512
NF = F // FBLK
T = N // TM
P = T + E
NP = P * TM
RB = 512
NB = N // RB
SQRT1_2 = 0.7071067811865476


def _router_body(x_ref, sw_ref, sb_ref, routes_ref, pmax_ref, rank_ref,
                 cnt_ref, xc_ref):
    xb = x_ref[...]
    logits = lax.dot_general(xb, sw_ref[...], (((1,), (1,)), ((), ())),
                             preferred_element_type=jnp.float32)
    logits = logits + sb_ref[0][None, :]
    m = jnp.max(logits, axis=-1, keepdims=True)
    denom = jnp.sum(jnp.exp(logits - m), axis=-1)
    pmax_ref[0, 0, :] = 1.0 / denom
    ids = lax.broadcasted_iota(jnp.int32, logits.shape, 1)
    routes = jnp.min(jnp.where(logits == m, ids, E), axis=-1)
    routes_ref[0, 0, :] = routes
    oh = (routes[:, None] == lax.broadcasted_iota(jnp.int32, (RB, E), 1))
    oh = oh.astype(jnp.float32)
    ti = lax.broadcasted_iota(jnp.int32, (RB, RB), 0)
    tj = lax.broadcasted_iota(jnp.int32, (RB, RB), 1)
    lt = (tj < ti).astype(jnp.float32)
    pre = lax.dot_general(lt, oh, (((1,), (0,)), ((), ())),
                          preferred_element_type=jnp.float32)
    rank_ref[0, 0, :] = jnp.sum(pre * oh, axis=1)
    cnt_ref[0, 0, :] = jnp.sum(oh, axis=0)
    xc_ref[...] = xb


def _slot_body(routes_ref, rank_ref, base_ref, slot_ref):
    r = routes_ref[0, 0, :]
    oh = (r[:, None] == lax.broadcasted_iota(jnp.int32, (RB, E), 1))
    base = jnp.sum(oh.astype(jnp.float32) * base_ref[0, 0, :][None, :], axis=1)
    slot_ref[0, 0, :] = (rank_ref[0, 0, :] + base).astype(jnp.int32)


def _ffn_body(eb_ref, x_ref, w1_ref, b1_ref, w2_ref, b2_ref, pr_ref, o_ref):
    xb = x_ref[...].astype(jnp.bfloat16)
    h = lax.dot_general(xb, w1_ref[0].astype(jnp.bfloat16),
                        (((1,), (1,)), ((), ())),
                        preferred_element_type=jnp.float32)
    h = h + b1_ref[0, 0][None, :]
    g = 0.5 * h * (1.0 + lax.erf(h * SQRT1_2))
    y = lax.dot_general(g.astype(jnp.bfloat16),
                        w2_ref[0].astype(jnp.bfloat16),
                        (((1,), (1,)), ((), ())),
                        preferred_element_type=jnp.float32)
    y = y + b2_ref[0, 0][None, :]
    o_ref[...] = y * pr_ref[0, 0][:, None]


def _route_tokens(flat, switch_w, switch_b):
    return pl.pallas_call(
        _router_body,
        grid=(NB,),
        in_specs=[
            pl.BlockSpec((RB, D), lambda i: (i, 0)),
            pl.BlockSpec((E, D), lambda i: (0, 0)),
            pl.BlockSpec((1, E), lambda i: (0, 0)),
        ],
        out_specs=[
            pl.BlockSpec((1, 1, RB), lambda i: (i, 0, 0)),
            pl.BlockSpec((1, 1, RB), lambda i: (i, 0, 0)),
            pl.BlockSpec((1, 1, RB), lambda i: (i, 0, 0)),
            pl.BlockSpec((1, 1, E), lambda i: (i, 0, 0)),
            pl.BlockSpec((RB, D), lambda i: (i, 0)),
        ],
        out_shape=[
            jax.ShapeDtypeStruct((NB, 1, RB), jnp.int32),
            jax.ShapeDtypeStruct((NB, 1, RB), jnp.float32),
            jax.ShapeDtypeStruct((NB, 1, RB), jnp.float32),
            jax.ShapeDtypeStruct((NB, 1, E), jnp.float32),
            jax.ShapeDtypeStruct((N, D), jnp.float32),
        ],
    )(flat, switch_w, switch_b.reshape(1, E))


def _token_slots(routes2, rank2, base):
    slot2 = pl.pallas_call(
        _slot_body,
        grid=(NB,),
        in_specs=[
            pl.BlockSpec((1, 1, RB), lambda i: (i, 0, 0)),
            pl.BlockSpec((1, 1, RB), lambda i: (i, 0, 0)),
            pl.BlockSpec((1, 1, E), lambda i: (i, 0, 0)),
        ],
        out_specs=pl.BlockSpec((1, 1, RB), lambda i: (i, 0, 0)),
        out_shape=jax.ShapeDtypeStruct((NB, 1, RB), jnp.int32),
    )(routes2, rank2, base)
    return slot2.reshape(N)


def _grouped_ffn(eblk, xg, w1, b1, w2, b2, pg2):
    grid_spec = pltpu.PrefetchScalarGridSpec(
        num_scalar_prefetch=1,
        grid=(P,),
        in_specs=[
            pl.BlockSpec((TM, D), lambda p, eb: (p, 0)),
            pl.BlockSpec((1, F, D), lambda p, eb: (eb[p], 0, 0)),
            pl.BlockSpec((1, 1, F), lambda p, eb: (eb[p], 0, 0)),
            pl.BlockSpec((1, D, F), lambda p, eb: (eb[p], 0, 0)),
            pl.BlockSpec((1, 1, D), lambda p, eb: (eb[p], 0, 0)),
            pl.BlockSpec((1, 1, TM), lambda p, eb: (p, 0, 0)),
        ],
        out_specs=pl.BlockSpec((TM, D), lambda p, eb: (p, 0)),
    )
    return pl.pallas_call(
        _ffn_body,
        grid_spec=grid_spec,
        out_shape=jax.ShapeDtypeStruct((NP, D), jnp.float32),
        compiler_params=pltpu.CompilerParams(
            dimension_semantics=("arbitrary",)),
    )(eblk, xg, w1, b1.reshape(E, 1, F), w2, b2.reshape(E, 1, D), pg2)


def _sc_gather_rows(src, idx, n_out):
    info = plsc.get_sparse_core_info()
    nc, ns = info.num_cores, info.num_subcores
    nw = nc * ns
    per_w = n_out // nw
    ch = 128
    mesh = plsc.VectorSubcoreMesh(core_axis_name="c", subcore_axis_name="s")

    @functools.partial(
        pl.kernel, mesh=mesh,
        out_type=jax.ShapeDtypeStruct((n_out, D), jnp.float32),
        scratch_types=[
            pltpu.VMEM((ch,), jnp.int32),
            pltpu.VMEM((ch, D), jnp.float32),
            pltpu.SemaphoreType.DMA,
        ],
    )
    def gk(src_hbm, idx_hbm, out_hbm, idx_v, rows_v, sem):
        wid = lax.axis_index("s") * nc + lax.axis_index("c")
        base0 = wid * per_w
        for c in range(per_w // ch):
            base = base0 + c * ch
            pltpu.sync_copy(idx_hbm.at[pl.ds(base, ch)], idx_v)
            pltpu.async_copy(src_hbm.at[idx_v], rows_v, sem).wait()
            pltpu.sync_copy(rows_v, out_hbm.at[pl.ds(base, ch)])

    return gk(src, idx)


def kernel(x, switch_w, switch_b, w1, b1, w2, b2):
    flat = x.reshape(N, D)

    routes2, pmax2, rank2, cnt2, xcopy = _route_tokens(flat, switch_w, switch_b)
    pmax = pmax2.reshape(N)

    cnt = cnt2.reshape(NB, E)
    cumpb = jnp.cumsum(cnt, axis=0) - cnt
    counts = cnt.sum(axis=0).astype(jnp.int32)
    ptiles = (counts + TM - 1) // TM
    bcum = jnp.cumsum(ptiles)
    pstart = (TM * (bcum - ptiles)).astype(jnp.float32)
    eblk = jnp.minimum(
        jnp.searchsorted(bcum, jnp.arange(P), side="right"), E - 1
    ).astype(jnp.int32)
    base = (pstart[None, :] + cumpb).reshape(NB, 1, E)

    slot = _token_slots(routes2, rank2, base)
    sidx = (jnp.arange(NP, dtype=jnp.int32) % N).at[slot].set(
        jnp.arange(N, dtype=jnp.int32), unique_indices=True,
        mode="promise_in_bounds")
    pgp = jnp.zeros(NP, jnp.float32).at[slot].set(
        pmax, unique_indices=True, mode="promise_in_bounds")

    xg = _sc_gather_rows(xcopy, sidx, NP)

    ys = _grouped_ffn(eblk, xg, w1, b1, w2, b2, pgp.reshape(P, 1, TM))

    final = _sc_gather_rows(ys, slot, N)
    return final.reshape(B, S, D)


# speedup vs baseline: 3.0364x; 1.0930x over previous
import functools

import jax
import jax.numpy as jnp
from jax import lax
from jax.experimental import pallas as pl
from jax.experimental.pallas import tpu as pltpu
from jax.experimental.pallas import tpu_sc as plsc

B, S, D, F, E = 2, 8192, 768, 2048, 64
N = B * S
TM = 128
FBLK = 512
NF = F // FBLK
T = N // TM
P = T + E
NP = P * TM
RB = 512
NB = N // RB
SQRT1_2 = 0.7071067811865476


def _router_body(x_ref, sw_ref, sb_ref, routes_ref, pmax_ref, rank_ref,
                 cnt_ref, xc_ref):
    xb = x_ref[...]
    logits = lax.dot_general(xb, sw_ref[...], (((1,), (1,)), ((), ())),
                             preferred_element_type=jnp.float32)
    logits = logits + sb_ref[0][None, :]
    m = jnp.max(logits, axis=-1, keepdims=True)
    denom = jnp.sum(jnp.exp(logits - m), axis=-1)
    pmax_ref[0, 0, :] = 1.0 / denom
    ids = lax.broadcasted_iota(jnp.int32, logits.shape, 1)
    routes = jnp.min(jnp.where(logits == m, ids, E), axis=-1)
    routes_ref[0, 0, :] = routes
    oh = (routes[:, None] == lax.broadcasted_iota(jnp.int32, (RB, E), 1))
    oh = oh.astype(jnp.float32)
    ti = lax.broadcasted_iota(jnp.int32, (RB, RB), 0)
    tj = lax.broadcasted_iota(jnp.int32, (RB, RB), 1)
    lt = (tj < ti).astype(jnp.float32)
    pre = lax.dot_general(lt, oh, (((1,), (0,)), ((), ())),
                          preferred_element_type=jnp.float32)
    rank_ref[0, 0, :] = jnp.sum(pre * oh, axis=1)
    cnt_ref[0, 0, :] = jnp.sum(oh, axis=0)
    xc_ref[...] = xb


def _slot_body(routes_ref, rank_ref, base_ref, slot_ref):
    r = routes_ref[0, 0, :]
    oh = (r[:, None] == lax.broadcasted_iota(jnp.int32, (RB, E), 1))
    base = jnp.sum(oh.astype(jnp.float32) * base_ref[0, 0, :][None, :], axis=1)
    slot_ref[0, 0, :] = (rank_ref[0, 0, :] + base).astype(jnp.int32)


def _ffn_body(eb_ref, x_ref, w1_ref, b1_ref, w2_ref, b2_ref, pr_ref, o_ref):
    xb = x_ref[...].astype(jnp.bfloat16)
    h = lax.dot_general(xb, w1_ref[0].astype(jnp.bfloat16),
                        (((1,), (1,)), ((), ())),
                        preferred_element_type=jnp.float32)
    h = h + b1_ref[0, 0][None, :]
    g = 0.5 * h * (1.0 + lax.erf(h * SQRT1_2))
    y = lax.dot_general(g.astype(jnp.bfloat16),
                        w2_ref[0].astype(jnp.bfloat16),
                        (((1,), (1,)), ((), ())),
                        preferred_element_type=jnp.float32)
    y = y + b2_ref[0, 0][None, :]
    o_ref[...] = y * pr_ref[0, 0][:, None]


def _route_tokens(flat, switch_w, switch_b):
    return pl.pallas_call(
        _router_body,
        grid=(NB,),
        in_specs=[
            pl.BlockSpec((RB, D), lambda i: (i, 0)),
            pl.BlockSpec((E, D), lambda i: (0, 0)),
            pl.BlockSpec((1, E), lambda i: (0, 0)),
        ],
        out_specs=[
            pl.BlockSpec((1, 1, RB), lambda i: (i, 0, 0)),
            pl.BlockSpec((1, 1, RB), lambda i: (i, 0, 0)),
            pl.BlockSpec((1, 1, RB), lambda i: (i, 0, 0)),
            pl.BlockSpec((1, 1, E), lambda i: (i, 0, 0)),
            pl.BlockSpec((RB, D), lambda i: (i, 0)),
        ],
        out_shape=[
            jax.ShapeDtypeStruct((NB, 1, RB), jnp.int32),
            jax.ShapeDtypeStruct((NB, 1, RB), jnp.float32),
            jax.ShapeDtypeStruct((NB, 1, RB), jnp.float32),
            jax.ShapeDtypeStruct((NB, 1, E), jnp.float32),
            jax.ShapeDtypeStruct((N, D), jnp.float32),
        ],
    )(flat, switch_w, switch_b.reshape(1, E))


def _token_slots(routes2, rank2, base):
    slot2 = pl.pallas_call(
        _slot_body,
        grid=(NB,),
        in_specs=[
            pl.BlockSpec((1, 1, RB), lambda i: (i, 0, 0)),
            pl.BlockSpec((1, 1, RB), lambda i: (i, 0, 0)),
            pl.BlockSpec((1, 1, E), lambda i: (i, 0, 0)),
        ],
        out_specs=pl.BlockSpec((1, 1, RB), lambda i: (i, 0, 0)),
        out_shape=jax.ShapeDtypeStruct((NB, 1, RB), jnp.int32),
    )(routes2, rank2, base)
    return slot2.reshape(N)


def _grouped_ffn(eblk, xg, w1, b1, w2, b2, pg2):
    grid_spec = pltpu.PrefetchScalarGridSpec(
        num_scalar_prefetch=1,
        grid=(P,),
        in_specs=[
            pl.BlockSpec((TM, D), lambda p, eb: (p, 0)),
            pl.BlockSpec((1, F, D), lambda p, eb: (eb[p], 0, 0)),
            pl.BlockSpec((1, 1, F), lambda p, eb: (eb[p], 0, 0)),
            pl.BlockSpec((1, D, F), lambda p, eb: (eb[p], 0, 0)),
            pl.BlockSpec((1, 1, D), lambda p, eb: (eb[p], 0, 0)),
            pl.BlockSpec((1, 1, TM), lambda p, eb: (p, 0, 0)),
        ],
        out_specs=pl.BlockSpec((TM, D), lambda p, eb: (p, 0)),
    )
    return pl.pallas_call(
        _ffn_body,
        grid_spec=grid_spec,
        out_shape=jax.ShapeDtypeStruct((NP, D), jnp.float32),
        compiler_params=pltpu.CompilerParams(
            dimension_semantics=("arbitrary",)),
    )(eblk, xg, w1, b1.reshape(E, 1, F), w2, b2.reshape(E, 1, D), pg2)


def _sc_scatter_tokens(flat, pmax, slot):
    info = plsc.get_sparse_core_info()
    nc, ns = info.num_cores, info.num_subcores
    nw = nc * ns
    per_w = N // nw
    ch = 128
    mesh = plsc.VectorSubcoreMesh(core_axis_name="c", subcore_axis_name="s")

    @functools.partial(
        pl.kernel, mesh=mesh,
        out_type=[
            jax.ShapeDtypeStruct((NP, D), jnp.float32),
            jax.ShapeDtypeStruct((NP,), jnp.float32),
        ],
        scratch_types=[
            pltpu.VMEM((ch,), jnp.int32),
            pltpu.VMEM((ch, D), jnp.float32),
            pltpu.VMEM((ch,), jnp.float32),
            pltpu.SemaphoreType.DMA,
        ],
    )
    def sk(flat_hbm, pmax_hbm, slot_hbm, xg_hbm, pgp_hbm,
           idx_v, rows_v, pbuf, sem):
        wid = lax.axis_index("s") * nc + lax.axis_index("c")
        base0 = wid * per_w
        for c in range(per_w // ch):
            base = base0 + c * ch
            pltpu.sync_copy(slot_hbm.at[pl.ds(base, ch)], idx_v)
            pltpu.sync_copy(flat_hbm.at[pl.ds(base, ch)], rows_v)
            pltpu.async_copy(rows_v, xg_hbm.at[idx_v], sem).wait()
            pltpu.sync_copy(pmax_hbm.at[pl.ds(base, ch)], pbuf)
            pltpu.async_copy(pbuf, pgp_hbm.at[idx_v], sem).wait()

    return sk(flat, pmax, slot)


def _sc_gather_rows(src, idx, n_out):
    info = plsc.get_sparse_core_info()
    nc, ns = info.num_cores, info.num_subcores
    nw = nc * ns
    per_w = n_out // nw
    ch = 128
    mesh = plsc.VectorSubcoreMesh(core_axis_name="c", subcore_axis_name="s")

    @functools.partial(
        pl.kernel, mesh=mesh,
        out_type=jax.ShapeDtypeStruct((n_out, D), jnp.float32),
        scratch_types=[
            pltpu.VMEM((ch,), jnp.int32),
            pltpu.VMEM((ch, D), jnp.float32),
            pltpu.SemaphoreType.DMA,
        ],
    )
    def gk(src_hbm, idx_hbm, out_hbm, idx_v, rows_v, sem):
        wid = lax.axis_index("s") * nc + lax.axis_index("c")
        base0 = wid * per_w
        for c in range(per_w // ch):
            base = base0 + c * ch
            pltpu.sync_copy(idx_hbm.at[pl.ds(base, ch)], idx_v)
            pltpu.async_copy(src_hbm.at[idx_v], rows_v, sem).wait()
            pltpu.sync_copy(rows_v, out_hbm.at[pl.ds(base, ch)])

    return gk(src, idx)


def kernel(x, switch_w, switch_b, w1, b1, w2, b2):
    flat = x.reshape(N, D)

    routes2, pmax2, rank2, cnt2, xcopy = _route_tokens(flat, switch_w, switch_b)
    pmax = pmax2.reshape(N)

    cnt = cnt2.reshape(NB, E)
    cumpb = jnp.cumsum(cnt, axis=0) - cnt
    counts = cnt.sum(axis=0).astype(jnp.int32)
    ptiles = (counts + TM - 1) // TM
    bcum = jnp.cumsum(ptiles)
    pstart = (TM * (bcum - ptiles)).astype(jnp.float32)
    eblk = jnp.minimum(
        jnp.searchsorted(bcum, jnp.arange(P), side="right"), E - 1
    ).astype(jnp.int32)
    base = (pstart[None, :] + cumpb).reshape(NB, 1, E)

    slot = _token_slots(routes2, rank2, base)

    xg, pgp = _sc_scatter_tokens(xcopy, pmax, slot)

    ys = _grouped_ffn(eblk, xg, w1, b1, w2, b2, pgp.reshape(P, 1, TM))

    final = _sc_gather_rows(ys, slot, N)
    return final.reshape(B, S, D)


# speedup vs baseline: 3.0498x; 1.0044x over previous
import functools

import jax
import jax.numpy as jnp
from jax import lax
from jax.experimental import pallas as pl
from jax.experimental.pallas import tpu as pltpu
from jax.experimental.pallas import tpu_sc as plsc

B, S, D, F, E = 2, 8192, 768, 2048, 64
N = B * S
TM = 128
FBLK = 512
NF = F // FBLK
T = N // TM
P = T + E
NP = P * TM
RB = 512
NB = N // RB
SQRT1_2 = 0.7071067811865476


def _router_body(x_ref, sw_ref, sb_ref, routes_ref, pmax_ref, rank_ref,
                 cnt_ref, xc_ref):
    xb = x_ref[...]
    logits = lax.dot_general(xb, sw_ref[...], (((1,), (1,)), ((), ())),
                             preferred_element_type=jnp.float32)
    logits = logits + sb_ref[0][None, :]
    m = jnp.max(logits, axis=-1, keepdims=True)
    denom = jnp.sum(jnp.exp(logits - m), axis=-1)
    pmax_ref[0, 0, :] = 1.0 / denom
    ids = lax.broadcasted_iota(jnp.int32, logits.shape, 1)
    routes = jnp.min(jnp.where(logits == m, ids, E), axis=-1)
    routes_ref[0, 0, :] = routes
    oh = (routes[:, None] == lax.broadcasted_iota(jnp.int32, (RB, E), 1))
    oh = oh.astype(jnp.float32)
    ti = lax.broadcasted_iota(jnp.int32, (RB, RB), 0)
    tj = lax.broadcasted_iota(jnp.int32, (RB, RB), 1)
    lt = (tj < ti).astype(jnp.float32)
    pre = lax.dot_general(lt, oh, (((1,), (0,)), ((), ())),
                          preferred_element_type=jnp.float32)
    rank_ref[0, 0, :] = jnp.sum(pre * oh, axis=1)
    cnt_ref[0, 0, :] = jnp.sum(oh, axis=0)
    xc_ref[...] = xb


def _slot_body(routes_ref, rank_ref, base_ref, slot_ref):
    r = routes_ref[0, 0, :]
    oh = (r[:, None] == lax.broadcasted_iota(jnp.int32, (RB, E), 1))
    base = jnp.sum(oh.astype(jnp.float32) * base_ref[0, 0, :][None, :], axis=1)
    slot_ref[0, 0, :] = (rank_ref[0, 0, :] + base).astype(jnp.int32)


def _ffn_body(eb_ref, x_ref, w1_ref, b1_ref, w2_ref, b2_ref, pr_ref, o_ref):
    xb = x_ref[...].astype(jnp.bfloat16)
    h = lax.dot_general(xb, w1_ref[0].astype(jnp.bfloat16),
                        (((1,), (1,)), ((), ())),
                        preferred_element_type=jnp.float32)
    h = h + b1_ref[0, 0][None, :]
    g = 0.5 * h * (1.0 + lax.erf(h * SQRT1_2))
    y = lax.dot_general(g.astype(jnp.bfloat16),
                        w2_ref[0].astype(jnp.bfloat16),
                        (((1,), (1,)), ((), ())),
                        preferred_element_type=jnp.float32)
    y = y + b2_ref[0, 0][None, :]
    o_ref[...] = y * pr_ref[0, 0][:, None]


def _route_tokens(flat, switch_w, switch_b):
    return pl.pallas_call(
        _router_body,
        grid=(NB,),
        in_specs=[
            pl.BlockSpec((RB, D), lambda i: (i, 0)),
            pl.BlockSpec((E, D), lambda i: (0, 0)),
            pl.BlockSpec((1, E), lambda i: (0, 0)),
        ],
        out_specs=[
            pl.BlockSpec((1, 1, RB), lambda i: (i, 0, 0)),
            pl.BlockSpec((1, 1, RB), lambda i: (i, 0, 0)),
            pl.BlockSpec((1, 1, RB), lambda i: (i, 0, 0)),
            pl.BlockSpec((1, 1, E), lambda i: (i, 0, 0)),
            pl.BlockSpec((RB, D), lambda i: (i, 0)),
        ],
        out_shape=[
            jax.ShapeDtypeStruct((NB, 1, RB), jnp.int32),
            jax.ShapeDtypeStruct((NB, 1, RB), jnp.float32),
            jax.ShapeDtypeStruct((NB, 1, RB), jnp.float32),
            jax.ShapeDtypeStruct((NB, 1, E), jnp.float32),
            jax.ShapeDtypeStruct((N, D), jnp.float32),
        ],
    )(flat, switch_w, switch_b.reshape(1, E))


def _token_slots(routes2, rank2, base):
    slot2 = pl.pallas_call(
        _slot_body,
        grid=(NB,),
        in_specs=[
            pl.BlockSpec((1, 1, RB), lambda i: (i, 0, 0)),
            pl.BlockSpec((1, 1, RB), lambda i: (i, 0, 0)),
            pl.BlockSpec((1, 1, E), lambda i: (i, 0, 0)),
        ],
        out_specs=pl.BlockSpec((1, 1, RB), lambda i: (i, 0, 0)),
        out_shape=jax.ShapeDtypeStruct((NB, 1, RB), jnp.int32),
    )(routes2, rank2, base)
    return slot2.reshape(N)


def _grouped_ffn(eblk, xg, w1, b1, w2, b2, pg2):
    grid_spec = pltpu.PrefetchScalarGridSpec(
        num_scalar_prefetch=1,
        grid=(P,),
        in_specs=[
            pl.BlockSpec((TM, D), lambda p, eb: (p, 0)),
            pl.BlockSpec((1, F, D), lambda p, eb: (eb[p], 0, 0)),
            pl.BlockSpec((1, 1, F), lambda p, eb: (eb[p], 0, 0)),
            pl.BlockSpec((1, D, F), lambda p, eb: (eb[p], 0, 0)),
            pl.BlockSpec((1, 1, D), lambda p, eb: (eb[p], 0, 0)),
            pl.BlockSpec((1, 1, TM), lambda p, eb: (p, 0, 0)),
        ],
        out_specs=pl.BlockSpec((TM, D), lambda p, eb: (p, 0)),
    )
    return pl.pallas_call(
        _ffn_body,
        grid_spec=grid_spec,
        out_shape=jax.ShapeDtypeStruct((NP, D), jnp.float32),
        compiler_params=pltpu.CompilerParams(
            dimension_semantics=("arbitrary",)),
    )(eblk, xg, w1, b1.reshape(E, 1, F), w2, b2.reshape(E, 1, D), pg2)


def _sc_scatter_tokens(flat, pmax, slot):
    info = plsc.get_sparse_core_info()
    nc, ns = info.num_cores, info.num_subcores
    nw = nc * ns
    per_w = N // nw
    ch = 128
    mesh = plsc.VectorSubcoreMesh(core_axis_name="c", subcore_axis_name="s")

    @functools.partial(
        pl.kernel, mesh=mesh,
        out_type=[
            jax.ShapeDtypeStruct((NP, D), jnp.float32),
            jax.ShapeDtypeStruct((NP,), jnp.float32),
        ],
        scratch_types=[
            pltpu.VMEM((ch,), jnp.int32),
            pltpu.VMEM((ch, D), jnp.float32),
            pltpu.VMEM((ch,), jnp.float32),
            pltpu.SemaphoreType.DMA,
        ],
    )
    def sk(flat_hbm, pmax_hbm, slot_hbm, xg_hbm, pgp_hbm,
           idx_v, rows_v, pbuf, sem):
        wid = lax.axis_index("s") * nc + lax.axis_index("c")
        base0 = wid * per_w
        for c in range(per_w // ch):
            base = base0 + c * ch
            pltpu.sync_copy(slot_hbm.at[pl.ds(base, ch)], idx_v)
            pltpu.sync_copy(flat_hbm.at[pl.ds(base, ch)], rows_v)
            pltpu.async_copy(rows_v, xg_hbm.at[idx_v], sem).wait()
            pltpu.sync_copy(pmax_hbm.at[pl.ds(base, ch)], pbuf)
            pltpu.async_copy(pbuf, pgp_hbm.at[idx_v], sem).wait()

    return sk(flat, pmax, slot)


def _sc_gather_rows(src, idx, n_out):
    info = plsc.get_sparse_core_info()
    nc, ns = info.num_cores, info.num_subcores
    nw = nc * ns
    per_w = n_out // nw
    ch = 128
    mesh = plsc.VectorSubcoreMesh(core_axis_name="c", subcore_axis_name="s")

    @functools.partial(
        pl.kernel, mesh=mesh,
        out_type=jax.ShapeDtypeStruct((n_out, D), jnp.float32),
        scratch_types=[
            pltpu.VMEM((ch,), jnp.int32),
            pltpu.VMEM((ch, D), jnp.float32),
            pltpu.SemaphoreType.DMA,
        ],
    )
    def gk(src_hbm, idx_hbm, out_hbm, idx_v, rows_v, sem):
        wid = lax.axis_index("s") * nc + lax.axis_index("c")
        base0 = wid * per_w
        for c in range(per_w // ch):
            base = base0 + c * ch
            pltpu.sync_copy(idx_hbm.at[pl.ds(base, ch)], idx_v)
            pltpu.async_copy(src_hbm.at[idx_v], rows_v, sem).wait()
            pltpu.sync_copy(rows_v, out_hbm.at[pl.ds(base, ch)])

    return gk(src, idx)


def kernel(x, switch_w, switch_b, w1, b1, w2, b2):
    flat = x.reshape(N, D)

    routes2, pmax2, rank2, cnt2, xcopy = _route_tokens(flat, switch_w, switch_b)
    pmax = pmax2.reshape(N)

    cnt = cnt2.reshape(NB, E)
    cumpb = jnp.cumsum(cnt, axis=0) - cnt
    counts = cnt.sum(axis=0).astype(jnp.int32)
    ptiles = (counts + TM - 1) // TM
    bcum = jnp.cumsum(ptiles)
    pstart = (TM * (bcum - ptiles)).astype(jnp.float32)
    eblk = jnp.minimum(
        (jnp.arange(P)[None, :] >= bcum[:, None]).sum(axis=0), E - 1
    ).astype(jnp.int32)
    base = (pstart[None, :] + cumpb).reshape(NB, 1, E)

    slot = _token_slots(routes2, rank2, base)

    xg, pgp = _sc_scatter_tokens(xcopy, pmax, slot)

    ys = _grouped_ffn(eblk, xg, w1, b1, w2, b2, pgp.reshape(P, 1, TM))

    final = _sc_gather_rows(ys, slot, N)
    return final.reshape(B, S, D)


# speedup vs baseline: 3.0530x; 1.0010x over previous
import functools

import jax
import jax.numpy as jnp
from jax import lax
from jax.experimental import pallas as pl
from jax.experimental.pallas import tpu as pltpu
from jax.experimental.pallas import tpu_sc as plsc

B, S, D, F, E = 2, 8192, 768, 2048, 64
N = B * S
TM = 128
FBLK = 512
NF = F // FBLK
T = N // TM
P = T + E
NP = P * TM
RB = 512
NB = N // RB
SQRT1_2 = 0.7071067811865476


def _router_body(x_ref, sw_ref, sb_ref, routes_ref, pmax_ref, rank_ref,
                 cnt_ref):
    xb = x_ref[...]
    logits = lax.dot_general(xb, sw_ref[...], (((1,), (1,)), ((), ())),
                             preferred_element_type=jnp.float32)
    logits = logits + sb_ref[0][None, :]
    m = jnp.max(logits, axis=-1, keepdims=True)
    denom = jnp.sum(jnp.exp(logits - m), axis=-1)
    pmax_ref[0, 0, :] = 1.0 / denom
    ids = lax.broadcasted_iota(jnp.int32, logits.shape, 1)
    routes = jnp.min(jnp.where(logits == m, ids, E), axis=-1)
    routes_ref[0, 0, :] = routes
    oh = (routes[:, None] == lax.broadcasted_iota(jnp.int32, (RB, E), 1))
    oh = oh.astype(jnp.float32)
    ti = lax.broadcasted_iota(jnp.int32, (RB, RB), 0)
    tj = lax.broadcasted_iota(jnp.int32, (RB, RB), 1)
    lt = (tj < ti).astype(jnp.float32)
    pre = lax.dot_general(lt, oh, (((1,), (0,)), ((), ())),
                          preferred_element_type=jnp.float32)
    rank_ref[0, 0, :] = jnp.sum(pre * oh, axis=1)
    cnt_ref[0, 0, :] = jnp.sum(oh, axis=0)


def _slot_body(routes_ref, rank_ref, base_ref, slot_ref):
    r = routes_ref[0, 0, :]
    oh = (r[:, None] == lax.broadcasted_iota(jnp.int32, (RB, E), 1))
    base = jnp.sum(oh.astype(jnp.float32) * base_ref[0, 0, :][None, :], axis=1)
    slot_ref[0, 0, :] = (rank_ref[0, 0, :] + base).astype(jnp.int32)


def _ffn_body(eb_ref, x_ref, w1_ref, b1_ref, w2_ref, b2_ref, pr_ref, o_ref):
    xb = x_ref[...].astype(jnp.bfloat16)
    h = lax.dot_general(xb, w1_ref[0].astype(jnp.bfloat16),
                        (((1,), (1,)), ((), ())),
                        preferred_element_type=jnp.float32)
    h = h + b1_ref[0, 0][None, :]
    g = 0.5 * h * (1.0 + lax.erf(h * SQRT1_2))
    y = lax.dot_general(g.astype(jnp.bfloat16),
                        w2_ref[0].astype(jnp.bfloat16),
                        (((1,), (1,)), ((), ())),
                        preferred_element_type=jnp.float32)
    y = y + b2_ref[0, 0][None, :]
    o_ref[...] = y * pr_ref[0, 0][:, None]


def _route_tokens(flat, switch_w, switch_b):
    return pl.pallas_call(
        _router_body,
        grid=(NB,),
        in_specs=[
            pl.BlockSpec((RB, D), lambda i: (i, 0)),
            pl.BlockSpec((E, D), lambda i: (0, 0)),
            pl.BlockSpec((1, E), lambda i: (0, 0)),
        ],
        out_specs=[
            pl.BlockSpec((1, 1, RB), lambda i: (i, 0, 0)),
            pl.BlockSpec((1, 1, RB), lambda i: (i, 0, 0)),
            pl.BlockSpec((1, 1, RB), lambda i: (i, 0, 0)),
            pl.BlockSpec((1, 1, E), lambda i: (i, 0, 0)),
        ],
        out_shape=[
            jax.ShapeDtypeStruct((NB, 1, RB), jnp.int32),
            jax.ShapeDtypeStruct((NB, 1, RB), jnp.float32),
            jax.ShapeDtypeStruct((NB, 1, RB), jnp.float32),
            jax.ShapeDtypeStruct((NB, 1, E), jnp.float32),
        ],
    )(flat, switch_w, switch_b.reshape(1, E))


def _token_slots(routes2, rank2, base):
    slot2 = pl.pallas_call(
        _slot_body,
        grid=(NB,),
        in_specs=[
            pl.BlockSpec((1, 1, RB), lambda i: (i, 0, 0)),
            pl.BlockSpec((1, 1, RB), lambda i: (i, 0, 0)),
            pl.BlockSpec((1, 1, E), lambda i: (i, 0, 0)),
        ],
        out_specs=pl.BlockSpec((1, 1, RB), lambda i: (i, 0, 0)),
        out_shape=jax.ShapeDtypeStruct((NB, 1, RB), jnp.int32),
    )(routes2, rank2, base)
    return slot2.reshape(N)


def _grouped_ffn(eblk, xg, w1, b1, w2, b2, pg2):
    grid_spec = pltpu.PrefetchScalarGridSpec(
        num_scalar_prefetch=1,
        grid=(P,),
        in_specs=[
            pl.BlockSpec((TM, D), lambda p, eb: (p, 0)),
            pl.BlockSpec((1, F, D), lambda p, eb: (eb[p], 0, 0)),
            pl.BlockSpec((1, 1, F), lambda p, eb: (eb[p], 0, 0)),
            pl.BlockSpec((1, D, F), lambda p, eb: (eb[p], 0, 0)),
            pl.BlockSpec((1, 1, D), lambda p, eb: (eb[p], 0, 0)),
            pl.BlockSpec((1, 1, TM), lambda p, eb: (p, 0, 0)),
        ],
        out_specs=pl.BlockSpec((TM, D), lambda p, eb: (p, 0)),
    )
    return pl.pallas_call(
        _ffn_body,
        grid_spec=grid_spec,
        out_shape=jax.ShapeDtypeStruct((NP, D), jnp.float32),
        compiler_params=pltpu.CompilerParams(
            dimension_semantics=("arbitrary",)),
    )(eblk, xg, w1, b1.reshape(E, 1, F), w2, b2.reshape(E, 1, D), pg2)


def _sc_scatter_tokens(flat, pmax, slot):
    info = plsc.get_sparse_core_info()
    nc, ns = info.num_cores, info.num_subcores
    nw = nc * ns
    per_w = N // nw
    ch = 128
    mesh = plsc.VectorSubcoreMesh(core_axis_name="c", subcore_axis_name="s")

    @functools.partial(
        pl.kernel, mesh=mesh,
        out_type=[
            jax.ShapeDtypeStruct((NP, D), jnp.float32),
            jax.ShapeDtypeStruct((NP,), jnp.float32),
        ],
        scratch_types=[
            pltpu.VMEM((ch,), jnp.int32),
            pltpu.VMEM((ch, D), jnp.float32),
            pltpu.VMEM((ch,), jnp.float32),
            pltpu.SemaphoreType.DMA,
        ],
    )
    def sk(flat_hbm, pmax_hbm, slot_hbm, xg_hbm, pgp_hbm,
           idx_v, rows_v, pbuf, sem):
        wid = lax.axis_index("s") * nc + lax.axis_index("c")
        base0 = wid * per_w
        for c in range(per_w // ch):
            base = base0 + c * ch
            pltpu.sync_copy(slot_hbm.at[pl.ds(base, ch)], idx_v)
            pltpu.sync_copy(flat_hbm.at[pl.ds(base, ch)], rows_v)
            pltpu.async_copy(rows_v, xg_hbm.at[idx_v], sem).wait()
            pltpu.sync_copy(pmax_hbm.at[pl.ds(base, ch)], pbuf)
            pltpu.async_copy(pbuf, pgp_hbm.at[idx_v], sem).wait()

    return sk(flat, pmax, slot)


def _sc_gather_rows(src, idx, n_out):
    info = plsc.get_sparse_core_info()
    nc, ns = info.num_cores, info.num_subcores
    nw = nc * ns
    per_w = n_out // nw
    ch = 128
    mesh = plsc.VectorSubcoreMesh(core_axis_name="c", subcore_axis_name="s")

    @functools.partial(
        pl.kernel, mesh=mesh,
        out_type=jax.ShapeDtypeStruct((n_out, D), jnp.float32),
        scratch_types=[
            pltpu.VMEM((ch,), jnp.int32),
            pltpu.VMEM((ch, D), jnp.float32),
            pltpu.SemaphoreType.DMA,
        ],
    )
    def gk(src_hbm, idx_hbm, out_hbm, idx_v, rows_v, sem):
        wid = lax.axis_index("s") * nc + lax.axis_index("c")
        base0 = wid * per_w
        for c in range(per_w // ch):
            base = base0 + c * ch
            pltpu.sync_copy(idx_hbm.at[pl.ds(base, ch)], idx_v)
            pltpu.async_copy(src_hbm.at[idx_v], rows_v, sem).wait()
            pltpu.sync_copy(rows_v, out_hbm.at[pl.ds(base, ch)])

    return gk(src, idx)


def kernel(x, switch_w, switch_b, w1, b1, w2, b2):
    flat = x.reshape(N, D)

    routes2, pmax2, rank2, cnt2 = _route_tokens(flat, switch_w, switch_b)
    pmax = pmax2.reshape(N)

    cnt = cnt2.reshape(NB, E)
    cumpb = jnp.cumsum(cnt, axis=0) - cnt
    counts = cnt.sum(axis=0).astype(jnp.int32)
    ptiles = (counts + TM - 1) // TM
    bcum = jnp.cumsum(ptiles)
    pstart = (TM * (bcum - ptiles)).astype(jnp.float32)
    eblk = jnp.minimum(
        (jnp.arange(P)[None, :] >= bcum[:, None]).sum(axis=0), E - 1
    ).astype(jnp.int32)
    base = (pstart[None, :] + cumpb).reshape(NB, 1, E)

    slot = _token_slots(routes2, rank2, base)

    xg, pgp = _sc_scatter_tokens(flat, pmax, slot)

    ys = _grouped_ffn(eblk, xg, w1, b1, w2, b2, pgp.reshape(P, 1, TM))

    final = _sc_gather_rows(ys, slot, N)
    return final.reshape(B, S, D)
